# baseline (speedup 1.0000x reference)
import functools

import jax
import jax.numpy as jnp
from jax import lax
from jax.experimental import pallas as pl
from jax.experimental.pallas import tpu as pltpu
from jax.experimental.pallas import tpu_sc as plsc

ROWS = 64
COLS = 32768
BINS = 1024
L = 16
NC = 2
NS = 16
NW = NC * NS
CHUNK = 16
NCH = 2
CW = COLS // NCH


def _zero_tbl(tbl_v):
    zeros = jnp.zeros((L,), jnp.float32)

    @plsc.parallel_loop(0, BINS, step=L, unroll=4)
    def zloop(i):
        tbl_v[pl.ds(i, L)] = zeros


def _hist(vals_v, tbl_v, base):
    ones = jnp.ones((L,), jnp.float32)

    @plsc.parallel_loop(base, base + CW, step=L, unroll=CHUNK)
    def hloop(i):
        v = vals_v[pl.ds(i, L)]
        plsc.addupdate_scatter(tbl_v, [v], ones)


def _invert(tbl_v):
    @plsc.parallel_loop(0, BINS, step=L, unroll=4)
    def iloop(i):
        tbl_v[pl.ds(i, L)] = 1.0 / tbl_v[pl.ds(i, L)]


def _gather(vals_v, tbl_v, out_v, base):
    @plsc.parallel_loop(base, base + CW, step=L, unroll=CHUNK)
    def gloop(i):
        v = vals_v[pl.ds(i, L)]
        out_v[pl.ds(i, L)] = plsc.load_gather(tbl_v, [v])


def _body(in_hbm, out_hbm, v0, v1, o_v, tbl_v, *sems):
    si0 = sems[0:NCH]
    si1 = sems[NCH:2 * NCH]
    so0 = sems[2 * NCH]
    so1 = sems[2 * NCH + 1]
    wid = lax.axis_index("s") * NC + lax.axis_index("c")
    row0 = wid * 2
    row1 = row0 + 1

    in0 = [
        pltpu.async_copy(
            in_hbm.at[row0, pl.ds(c * CW, CW)], v0.at[pl.ds(c * CW, CW)], si0[c]
        )
        for c in range(NCH)
    ]
    in1 = [
        pltpu.async_copy(
            in_hbm.at[row1, pl.ds(c * CW, CW)], v1.at[pl.ds(c * CW, CW)], si1[c]
        )
        for c in range(NCH)
    ]

    _zero_tbl(tbl_v)
    for c in range(NCH):
        in0[c].wait()
        _hist(v0, tbl_v, c * CW)
    _invert(tbl_v)

    out0 = []
    for c in range(NCH):
        _gather(v0, tbl_v, o_v, c * CW)
        out0.append(
            pltpu.async_copy(
                o_v.at[pl.ds(c * CW, CW)], out_hbm.at[row0, pl.ds(c * CW, CW)], so0
            )
        )

    _zero_tbl(tbl_v)
    for c in range(NCH):
        in1[c].wait()
        _hist(v1, tbl_v, c * CW)
    _invert(tbl_v)

    for c in range(NCH):
        out0[c].wait()

    out1 = []
    for c in range(NCH):
        _gather(v1, tbl_v, o_v, c * CW)
        out1.append(
            pltpu.async_copy(
                o_v.at[pl.ds(c * CW, CW)], out_hbm.at[row1, pl.ds(c * CW, CW)], so1
            )
        )
    for c in range(NCH):
        out1[c].wait()


@jax.jit
def kernel(inputs):
    k = pl.kernel(
        _body,
        out_type=jax.ShapeDtypeStruct((ROWS, COLS), jnp.float32),
        mesh=plsc.VectorSubcoreMesh(core_axis_name="c", subcore_axis_name="s"),
        scratch_types=[
            pltpu.VMEM((COLS,), jnp.int32),
            pltpu.VMEM((COLS,), jnp.int32),
            pltpu.VMEM((COLS,), jnp.float32),
            pltpu.VMEM((BINS,), jnp.float32),
        ]
        + [pltpu.SemaphoreType.DMA] * (2 * NCH + 2),
        compiler_params=pltpu.CompilerParams(needs_layout_passes=False),
    )
    return k(inputs.astype(jnp.int32))


# speedup vs baseline: 1.0017x; 1.0017x over previous
import jax
import jax.numpy as jnp
from jax import lax
from jax.experimental import pallas as pl
from jax.experimental.pallas import tpu as pltpu
from jax.experimental.pallas import tpu_sc as plsc

ROWS = 64
COLS = 32768
BINS = 1024
L = 16
NC = 2
NS = 16
NW = NC * NS
CHUNK = 16
NCH = 2
CW = COLS // NCH


def _zero_tbl(tbl_v):
    zeros = jnp.zeros((L,), jnp.float32)

    @plsc.parallel_loop(0, BINS, step=L, unroll=4)
    def zloop(i):
        tbl_v[pl.ds(i, L)] = zeros


def _hist(vals_v, tbl_v, base):
    ones = jnp.ones((L,), jnp.float32)

    @plsc.parallel_loop(base, base + CW, step=L, unroll=CHUNK)
    def hloop(i):
        v = vals_v[pl.ds(i, L)]
        plsc.addupdate_scatter(tbl_v, [v], ones)


def _invert(tbl_v):
    @plsc.parallel_loop(0, BINS, step=L, unroll=4)
    def iloop(i):
        tbl_v[pl.ds(i, L)] = 1.0 / tbl_v[pl.ds(i, L)]


def _gather(vals_v, tbl_v, out_v, base):
    @plsc.parallel_loop(base, base + CW, step=L, unroll=CHUNK)
    def gloop(i):
        v = vals_v[pl.ds(i, L)]
        out_v[pl.ds(i, L)] = plsc.load_gather(tbl_v, [v])


def _body(in_hbm, out_hbm, v0, v1, o_v, tbl_v, *sems):
    si0 = sems[0:NCH]
    si1 = sems[NCH:2 * NCH]
    so0 = sems[2 * NCH]
    so1 = sems[2 * NCH + 1]
    wid = lax.axis_index("s") * NC + lax.axis_index("c")
    row0 = wid * 2
    row1 = row0 + 1

    in0 = [
        pltpu.async_copy(
            in_hbm.at[row0, pl.ds(c * CW, CW)], v0.at[pl.ds(c * CW, CW)], si0[c]
        )
        for c in range(NCH)
    ]
    in1 = [
        pltpu.async_copy(
            in_hbm.at[row1, pl.ds(c * CW, CW)], v1.at[pl.ds(c * CW, CW)], si1[c]
        )
        for c in range(NCH)
    ]

    _zero_tbl(tbl_v)
    for c in range(NCH):
        in0[c].wait()
        _hist(v0, tbl_v, c * CW)
    _invert(tbl_v)

    out0 = []
    for c in range(NCH):
        _gather(v0, tbl_v, o_v, c * CW)
        out0.append(
            pltpu.async_copy(
                o_v.at[pl.ds(c * CW, CW)], out_hbm.at[row0, pl.ds(c * CW, CW)], so0
            )
        )

    _zero_tbl(tbl_v)
    for c in range(NCH):
        in1[c].wait()
        _hist(v1, tbl_v, c * CW)
    _invert(tbl_v)

    for c in range(NCH):
        out0[c].wait()

    out1 = []
    for c in range(NCH):
        _gather(v1, tbl_v, o_v, c * CW)
        out1.append(
            pltpu.async_copy(
                o_v.at[pl.ds(c * CW, CW)], out_hbm.at[row1, pl.ds(c * CW, CW)], so1
            )
        )
    for c in range(NCH):
        out1[c].wait()


@jax.jit
def kernel(inputs):
    k = pl.kernel(
        _body,
        out_type=jax.ShapeDtypeStruct((ROWS, COLS), jnp.float32),
        mesh=plsc.VectorSubcoreMesh(core_axis_name="c", subcore_axis_name="s"),
        scratch_types=[
            pltpu.VMEM((COLS,), jnp.int32),
            pltpu.VMEM((COLS,), jnp.int32),
            pltpu.VMEM((COLS,), jnp.float32),
            pltpu.VMEM((BINS,), jnp.float32),
        ]
        + [pltpu.SemaphoreType.DMA] * (2 * NCH + 2),
        compiler_params=pltpu.CompilerParams(needs_layout_passes=False),
    )
    return k(inputs.astype(jnp.int32))
